# baseline (speedup 1.0000x reference)
import jax
import jax.numpy as jnp
from jax import lax
from jax.experimental import pallas as pl
from jax.experimental.pallas import tpu as pltpu
from jax.experimental.pallas import tpu_sc as plsc

_B = 16384
_F = 26
_FH = _F // 2
_V = 100000
_D = 16
_HID = 256
_OUT = 512

_NH = _B * _FH
_NC = 2
_NS = 16
_NW = _NC * _NS
_BPW = _NH // _NW
_CHUNK = 1664
_NCHUNK = _BPW // _CHUNK


def _gather_body(tab, idx_hbm, out_hbm, idx_v, rows_v, sem):
    wid = lax.axis_index("s") * _NC + lax.axis_index("c")
    base = wid * _BPW

    def body(c, carry):
        off = base + c * _CHUNK
        pltpu.sync_copy(idx_hbm.at[pl.ds(off, _CHUNK)], idx_v)
        pltpu.async_copy(tab.at[idx_v], rows_v, sem).wait()
        pltpu.sync_copy(rows_v, out_hbm.at[pl.ds(off, _CHUNK)])
        return carry

    lax.fori_loop(0, _NCHUNK, body, 0)


_sc_gather = pl.kernel(
    _gather_body,
    out_type=jax.ShapeDtypeStruct((_NH, _D), jnp.float32),
    mesh=plsc.VectorSubcoreMesh(core_axis_name="c", subcore_axis_name="s"),
    scratch_types=[
        pltpu.VMEM((_CHUNK,), jnp.int32),
        pltpu.VMEM((_CHUNK, _D), jnp.float32),
        pltpu.SemaphoreType.DMA,
    ],
    compiler_params=pltpu.CompilerParams(use_tc_tiling_on_sc=False),
)


_BM = 1024


def _mlp_body(ea_ref, eb_ref, w1a_ref, w1b_ref, b1_ref, w2_ref, b2_ref,
              out_ref):
    h = (jnp.dot(ea_ref[...], w1a_ref[...],
                 preferred_element_type=jnp.float32) +
         jnp.dot(eb_ref[...], w1b_ref[...],
                 preferred_element_type=jnp.float32) + b1_ref[...])
    o = jnp.dot(h, w2_ref[...],
                preferred_element_type=jnp.float32) + b2_ref[...]
    out_ref[...] = jnp.maximum(o, 0.0)


def _tc_mlp(embA, embB, W1, b1, W2, b2):
    _HF = _FH * _D
    return pl.pallas_call(
        _mlp_body,
        grid=(_B // _BM,),
        in_specs=[
            pl.BlockSpec((_BM, _HF), lambda i: (i, 0)),
            pl.BlockSpec((_BM, _HF), lambda i: (i, 0)),
            pl.BlockSpec((_HF, _HID), lambda i: (0, 0)),
            pl.BlockSpec((_HF, _HID), lambda i: (0, 0)),
            pl.BlockSpec((1, _HID), lambda i: (0, 0)),
            pl.BlockSpec((_HID, _OUT), lambda i: (0, 0)),
            pl.BlockSpec((1, _OUT), lambda i: (0, 0)),
        ],
        out_specs=pl.BlockSpec((_BM, _OUT), lambda i: (i, 0)),
        out_shape=jax.ShapeDtypeStruct((_B, _OUT), jnp.float32),
    )(embA, embB, W1[:_HF], W1[_HF:], b1.reshape(1, _HID), W2,
      b2.reshape(1, _OUT))


def kernel(x, tables, W1, b1, W2, b2):
    xi = x.astype(jnp.int32)
    offs = (jnp.arange(_FH, dtype=jnp.int32) * _V)[None, :]
    idxA = (xi[:, :_FH] + offs).reshape(-1)
    idxB = (xi[:, _FH:] + offs).reshape(-1)
    tabA = tables[:_FH].reshape(_FH * _V, _D)
    tabB = tables[_FH:].reshape(_FH * _V, _D)
    embA = _sc_gather(tabA, idxA)
    embB = _sc_gather(tabB, idxB)
    return _tc_mlp(embA.reshape(_B, _FH * _D), embB.reshape(_B, _FH * _D),
                   W1, b1, W2, b2)


# speedup vs baseline: 1.4731x; 1.4731x over previous
import jax
import jax.numpy as jnp
from jax import lax
from jax.experimental import pallas as pl
from jax.experimental.pallas import tpu as pltpu
from jax.experimental.pallas import tpu_sc as plsc

_B = 16384
_F = 26
_V = 100000
_D = 16
_HID = 256
_OUT = 512

_N = _B * _F
_NC = 2
_NS = 16
_NW = _NC * _NS
_BPW = _N // _NW
_CHUNK = 1664
_NCHUNK = _BPW // _CHUNK


def _gather_body(tab, idx_hbm, out_hbm, idx_v, rows_v, sem):
    wid = lax.axis_index("s") * _NC + lax.axis_index("c")
    base = wid * _BPW

    def body(c, carry):
        off = base + c * _CHUNK
        pltpu.sync_copy(idx_hbm.at[pl.ds(off, _CHUNK)], idx_v)
        pltpu.async_copy(tab.at[idx_v], rows_v, sem).wait()
        pltpu.sync_copy(rows_v, out_hbm.at[pl.ds(off, _CHUNK)])
        return carry

    lax.fori_loop(0, _NCHUNK, body, 0)


_sc_gather = pl.kernel(
    _gather_body,
    out_type=jax.ShapeDtypeStruct((_N, _D), jnp.float32),
    mesh=plsc.VectorSubcoreMesh(core_axis_name="c", subcore_axis_name="s"),
    scratch_types=[
        pltpu.VMEM((_CHUNK,), jnp.int32),
        pltpu.VMEM((_CHUNK, _D), jnp.float32),
        pltpu.SemaphoreType.DMA,
    ],
    compiler_params=pltpu.CompilerParams(use_tc_tiling_on_sc=False),
)


_BM = 1024


def _mlp_body(emb_ref, w1_ref, b1_ref, w2_ref, b2_ref, out_ref):
    h = jnp.dot(emb_ref[...], w1_ref[...],
                preferred_element_type=jnp.float32) + b1_ref[...]
    o = jnp.dot(h, w2_ref[...],
                preferred_element_type=jnp.float32) + b2_ref[...]
    out_ref[...] = jnp.maximum(o, 0.0)


def _tc_mlp(emb, W1, b1, W2, b2):
    return pl.pallas_call(
        _mlp_body,
        grid=(_B // _BM,),
        in_specs=[
            pl.BlockSpec((_BM, _F * _D), lambda i: (i, 0)),
            pl.BlockSpec((_F * _D, _HID), lambda i: (0, 0)),
            pl.BlockSpec((1, _HID), lambda i: (0, 0)),
            pl.BlockSpec((_HID, _OUT), lambda i: (0, 0)),
            pl.BlockSpec((1, _OUT), lambda i: (0, 0)),
        ],
        out_specs=pl.BlockSpec((_BM, _OUT), lambda i: (i, 0)),
        out_shape=jax.ShapeDtypeStruct((_B, _OUT), jnp.float32),
    )(emb, W1, b1.reshape(1, _HID), W2, b2.reshape(1, _OUT))


def kernel(x, tables, W1, b1, W2, b2):
    xi = x.astype(jnp.int32)
    offs = (jnp.arange(_F, dtype=jnp.int32) * _V)[None, :]
    idx = (xi + offs).reshape(-1)
    emb = _sc_gather(tables.reshape(_F * _V, _D), idx)
    return _tc_mlp(emb.reshape(_B, _F * _D), W1, b1, W2, b2)
